# baseline (speedup 1.0000x reference)
import functools

import jax
import jax.numpy as jnp
from jax import lax
from jax.experimental import pallas as pl
from jax.experimental.pallas import tpu as pltpu
from jax.experimental.pallas import tpu_sc as plsc

_BATCH, _SEQ, _DIM = 4096, 200, 128
_B = _BATCH * _SEQ
_NC, _NS = 2, 16
_NW = _NC * _NS
_BPW = _B // _NW
_C = 128
_NCHUNK = _BPW // _C


_NBUF = 4
_NGROUP = _NCHUNK // _NBUF


def _gather_body(table_hbm, idx_hbm, out_hbm, idx_v, rows_v, gsems, wsems):
    wid = lax.axis_index("s") * _NC + lax.axis_index("c")
    base = wid * _BPW
    pltpu.sync_copy(idx_hbm.at[wid], idx_v)

    def gather(j, s):
        return pltpu.async_copy(table_hbm.at[idx_v.at[j]], rows_v.at[s], gsems.at[s])

    def writeback(j, s):
        return pltpu.async_copy(
            rows_v.at[s], out_hbm.at[pl.ds(base + j * _C, _C)], wsems.at[s])

    for s in range(_NBUF):
        gather(s, s)

    def group(i, carry):
        j = i * _NBUF
        for s in range(_NBUF):
            pltpu.make_async_copy(
                table_hbm.at[idx_v.at[j + s]], rows_v.at[s], gsems.at[s]).wait()
            writeback(j + s, s)
        @pl.when(i + 1 < _NGROUP)
        def _():
            for s in range(_NBUF):
                pltpu.make_async_copy(
                    rows_v.at[s],
                    out_hbm.at[pl.ds(base + (j + s) * _C, _C)],
                    wsems.at[s]).wait()
                gather(j + s + _NBUF, s)
        return carry

    lax.fori_loop(0, _NGROUP, group, 0)

    jlast = (_NGROUP - 1) * _NBUF
    for s in range(_NBUF):
        pltpu.make_async_copy(
            rows_v.at[s],
            out_hbm.at[pl.ds(base + (jlast + s) * _C, _C)],
            wsems.at[s]).wait()


_gather = pl.kernel(
    _gather_body,
    out_type=jax.ShapeDtypeStruct((_B, _DIM), jnp.float32),
    mesh=plsc.VectorSubcoreMesh(core_axis_name="c", subcore_axis_name="s"),
    scratch_types=[
        pltpu.VMEM((_NCHUNK, _C), jnp.int32),
        pltpu.VMEM((_NBUF, _C, _DIM), jnp.float32),
        pltpu.SemaphoreType.DMA((_NBUF,)),
        pltpu.SemaphoreType.DMA((_NBUF,)),
    ],
)


@jax.jit
def _run(x, table):
    idx = x.reshape(_NW, _NCHUNK, _C).astype(jnp.int32)
    out = _gather(table, idx)
    return out.reshape(_BATCH, _SEQ, _DIM)


def kernel(x, table):
    return _run(x, table)


# speedup vs baseline: 1.0119x; 1.0119x over previous
import functools

import jax
import jax.numpy as jnp
from jax import lax
from jax.experimental import pallas as pl
from jax.experimental.pallas import tpu as pltpu
from jax.experimental.pallas import tpu_sc as plsc

_BATCH, _SEQ, _DIM = 4096, 200, 128
_B = _BATCH * _SEQ
_NC, _NS = 2, 16
_NW = _NC * _NS
_BPW = _B // _NW
_C = 128
_NCHUNK = _BPW // _C
_NPAIR = _NCHUNK // 2


def _gather_body(table_hbm, idx_hbm, out_hbm, idx_v, rows_v, gsem0, gsem1):
    wid = lax.axis_index("s") * _NC + lax.axis_index("c")
    base = wid * _BPW
    pltpu.sync_copy(idx_hbm.at[wid], idx_v)

    pltpu.async_copy(table_hbm.at[idx_v.at[0]], rows_v.at[0], gsem0)

    def pair(i, carry):
        j0 = i * 2
        j1 = j0 + 1
        pltpu.async_copy(table_hbm.at[idx_v.at[j1]], rows_v.at[1], gsem1)
        pltpu.make_async_copy(table_hbm.at[idx_v.at[j0]], rows_v.at[0], gsem0).wait()
        pltpu.sync_copy(rows_v.at[0], out_hbm.at[pl.ds(base + j0 * _C, _C)])
        @pl.when(i + 1 < _NPAIR)
        def _():
            pltpu.async_copy(table_hbm.at[idx_v.at[j0 + 2]], rows_v.at[0], gsem0)
        pltpu.make_async_copy(table_hbm.at[idx_v.at[j1]], rows_v.at[1], gsem1).wait()
        pltpu.sync_copy(rows_v.at[1], out_hbm.at[pl.ds(base + j1 * _C, _C)])
        return carry

    lax.fori_loop(0, _NPAIR, pair, 0)


_gather = pl.kernel(
    _gather_body,
    out_type=jax.ShapeDtypeStruct((_B, _DIM), jnp.float32),
    mesh=plsc.VectorSubcoreMesh(core_axis_name="c", subcore_axis_name="s"),
    scratch_types=[
        pltpu.VMEM((_NCHUNK, _C), jnp.int32),
        pltpu.VMEM((2, _C, _DIM), jnp.float32),
        pltpu.SemaphoreType.DMA,
        pltpu.SemaphoreType.DMA,
    ],
)


@jax.jit
def _run(x, table):
    idx = x.reshape(_NW, _NCHUNK, _C).astype(jnp.int32)
    out = _gather(table, idx)
    return out.reshape(_BATCH, _SEQ, _DIM)


def kernel(x, table):
    return _run(x, table)


# speedup vs baseline: 1.0123x; 1.0004x over previous
import jax
import jax.numpy as jnp
from jax import lax
from jax.experimental import pallas as pl
from jax.experimental.pallas import tpu as pltpu
from jax.experimental.pallas import tpu_sc as plsc

_BATCH, _SEQ, _DIM = 4096, 200, 128
_B = _BATCH * _SEQ
_NC, _NS = 2, 16
_NW = _NC * _NS
_BPW = _B // _NW
_C = 128
_NCHUNK = _BPW // _C
_NPAIR = _NCHUNK // 2


def _gather_body(table_hbm, idx_hbm, out_hbm, idx_v, rows_v, gsem0, gsem1):
    wid = lax.axis_index("s") * _NC + lax.axis_index("c")
    base = wid * _BPW
    pltpu.sync_copy(idx_hbm.at[wid], idx_v)

    pltpu.async_copy(table_hbm.at[idx_v.at[0]], rows_v.at[0], gsem0)

    def pair(i, carry):
        j0 = i * 2
        j1 = j0 + 1
        pltpu.async_copy(table_hbm.at[idx_v.at[j1]], rows_v.at[1], gsem1, priority=1)
        pltpu.make_async_copy(table_hbm.at[idx_v.at[j0]], rows_v.at[0], gsem0).wait()
        pltpu.sync_copy(rows_v.at[0], out_hbm.at[pl.ds(base + j0 * _C, _C)])
        @pl.when(i + 1 < _NPAIR)
        def _():
            pltpu.async_copy(table_hbm.at[idx_v.at[j0 + 2]], rows_v.at[0], gsem0, priority=1)
        pltpu.make_async_copy(table_hbm.at[idx_v.at[j1]], rows_v.at[1], gsem1).wait()
        pltpu.sync_copy(rows_v.at[1], out_hbm.at[pl.ds(base + j1 * _C, _C)])
        return carry

    lax.fori_loop(0, _NPAIR, pair, 0)


_gather = pl.kernel(
    _gather_body,
    out_type=jax.ShapeDtypeStruct((_B, _DIM), jnp.float32),
    mesh=plsc.VectorSubcoreMesh(core_axis_name="c", subcore_axis_name="s"),
    scratch_types=[
        pltpu.VMEM((_NCHUNK, _C), jnp.int32),
        pltpu.VMEM((2, _C, _DIM), jnp.float32),
        pltpu.SemaphoreType.DMA,
        pltpu.SemaphoreType.DMA,
    ],
)


@jax.jit
def _run(x, table):
    idx = x.reshape(_NW, _NCHUNK, _C).astype(jnp.int32)
    out = _gather(table, idx)
    return out.reshape(_BATCH, _SEQ, _DIM)


def kernel(x, table):
    return _run(x, table)


# speedup vs baseline: 1.0140x; 1.0017x over previous
import jax
import jax.numpy as jnp
from jax import lax
from jax.experimental import pallas as pl
from jax.experimental.pallas import tpu as pltpu
from jax.experimental.pallas import tpu_sc as plsc

_BATCH, _SEQ, _DIM = 4096, 200, 128
_B = _BATCH * _SEQ
_NC, _NS = 2, 16
_NW = _NC * _NS
_BPW = _B // _NW
_C = 128
_NCHUNK = _BPW // _C
_NPAIR = _NCHUNK // 2


def _gather_body(table_hbm, idx_hbm, out_hbm, idx_v, rows_v, gsem0, gsem1):
    wid = lax.axis_index("s") * _NC + lax.axis_index("c")
    base = wid * _BPW
    pltpu.sync_copy(idx_hbm.at[wid], idx_v)

    pltpu.async_copy(table_hbm.at[idx_v.at[0]], rows_v.at[0], gsem0)

    def pair(i, carry):
        j0 = i * 2
        j1 = j0 + 1
        pltpu.async_copy(table_hbm.at[idx_v.at[j1]], rows_v.at[1], gsem1)
        pltpu.make_async_copy(table_hbm.at[idx_v.at[j0]], rows_v.at[0], gsem0).wait()
        pltpu.sync_copy(rows_v.at[0], out_hbm.at[pl.ds(base + j0 * _C, _C)])
        @pl.when(i + 1 < _NPAIR)
        def _():
            pltpu.async_copy(table_hbm.at[idx_v.at[j0 + 2]], rows_v.at[0], gsem0)
        pltpu.make_async_copy(table_hbm.at[idx_v.at[j1]], rows_v.at[1], gsem1).wait()
        pltpu.sync_copy(rows_v.at[1], out_hbm.at[pl.ds(base + j1 * _C, _C)])
        return carry

    lax.fori_loop(0, _NPAIR, pair, 0)


_gather = pl.kernel(
    _gather_body,
    out_type=jax.ShapeDtypeStruct((_B, _DIM), jnp.float32),
    mesh=plsc.VectorSubcoreMesh(core_axis_name="c", subcore_axis_name="s"),
    scratch_types=[
        pltpu.VMEM((_NCHUNK, _C), jnp.int32),
        pltpu.VMEM((2, _C, _DIM), jnp.float32),
        pltpu.SemaphoreType.DMA,
        pltpu.SemaphoreType.DMA,
    ],
)


@jax.jit
def _run(x, table):
    idx = x.reshape(_NW, _NCHUNK, _C).astype(jnp.int32)
    out = _gather(table, idx)
    return out.reshape(_BATCH, _SEQ, _DIM)


def kernel(x, table):
    return _run(x, table)
